# baseline (speedup 1.0000x reference)
import functools

import jax
import jax.numpy as jnp
from jax import lax
from jax.experimental import pallas as pl
from jax.experimental.pallas import tpu as pltpu
from jax.experimental.pallas import tpu_sc as plsc

_NC = 2
_NSC = 16
_NW = _NC * _NSC
_LANES = 16

_BLK = 400
_SPLIT = 5600
_TNODES = 8
_NBUF = 3
_KSPLIT = 2


def _sc_probe_body(nb_hbm, nsum_hbm, buf0, buf1, buf2, sem0, sem1, sem2,
                   *, t_nodes, ns, d, n, ch0):
    nch = n // t_nodes
    rows = t_nodes * ns
    count = nch - ch0
    niter = (count + _NW - 1) // _NW
    niter += (-niter) % _NBUF
    bufs = (buf0, buf1, buf2)
    sems = (sem0, sem1, sem2)
    sid = lax.axis_index("s")
    wid = lax.axis_index("c") * _NSC + sid

    def start(c, b):
        pltpu.async_copy(nb_hbm.at[pl.ds(c * rows, rows)],
                         bufs[b].at[pl.ds(sid * rows, rows)], sems[b])

    def wait(c, b):
        pltpu.make_async_copy(nb_hbm.at[pl.ds(c * rows, rows)],
                              bufs[b].at[pl.ds(sid * rows, rows)],
                              sems[b]).wait()

    for b in range(_NBUF - 1):
        c = ch0 + wid + _NW * b

        @pl.when(c < nch)
        def _(c=c, b=b):
            start(c, b)

    def outer(i, carry):
        for b in range(_NBUF):
            c = ch0 + wid + _NW * (_NBUF * i + b)

            @pl.when(c < nch)
            def _(c=c, b=b):
                wait(c, b)
                cn = c + _NW * (_NBUF - 1)

                @pl.when(cn < nch)
                def _():
                    start(cn, (b + _NBUF - 1) % _NBUF)

        return carry

    lax.fori_loop(0, niter // _NBUF, outer, 0)


def _sc_probe(neighbor_feats, n, ns, d, ch0, t_nodes):
    body = functools.partial(_sc_probe_body, t_nodes=t_nodes, ns=ns, d=d,
                             n=n, ch0=ch0)
    mesh = plsc.VectorSubcoreMesh(core_axis_name="c", subcore_axis_name="s")
    rows = t_nodes * ns
    return pl.kernel(
        body,
        out_type=jax.ShapeDtypeStruct((n - ch0 * t_nodes, d), jnp.float32),
        mesh=mesh,
        scratch_types=(
            [pltpu.VMEM_SHARED((_NSC * rows, d), jnp.float32)
             for _ in range(_NBUF)]
            + [pltpu.SemaphoreType.DMA for _ in range(_NBUF)]
        ),
    )(neighbor_feats)


def _sc_segsum_body(nb_hbm, nsum_hbm, *refs, t_nodes, ns, d, n, ch0):
    bufs = refs[:_NBUF]
    obufs = refs[_NBUF:2 * _NBUF]
    sems = refs[2 * _NBUF:3 * _NBUF]
    osems = refs[3 * _NBUF:4 * _NBUF]
    nch = n // t_nodes
    rows = t_nodes * ns
    ncol = d // _LANES
    count = nch - ch0
    assert count >= _NW * _NBUF
    assert rows % _KSPLIT == 0 and (rows // _KSPLIT) % 8 == 0
    niter = (count + _NW - 1) // _NW
    niter += (-niter) % _NBUF
    wid = lax.axis_index("c") * _NSC + lax.axis_index("s")

    half = rows // _KSPLIT

    def in_copy(c, b):
        for h in range(_KSPLIT):
            pltpu.async_copy(
                nb_hbm.at[pl.ds(c * rows + h * half, half)],
                bufs[b].at[pl.ds(h * half, half)], sems[b])

    for b in range(_NBUF - 1):
        c = ch0 + wid + _NW * b

        @pl.when(c < nch)
        def _(c=c, b=b):
            in_copy(c, b)

    def outer(i, carry):
        for b in range(_NBUF):
            idx = _NBUF * i + b
            c = ch0 + wid + _NW * idx

            @pl.when(c < nch)
            def _(c=c, b=b):
                pltpu.make_async_copy(
                    nb_hbm.at[pl.ds(c * rows, rows)], bufs[b], sems[b]).wait()
                cn = c + _NW * (_NBUF - 1)

                @pl.when(cn < nch)
                def _():
                    in_copy(cn, (b + _NBUF - 1) % _NBUF)

                @pl.when(i >= 1)
                def _():
                    pltpu.make_async_copy(
                        obufs[b],
                        nsum_hbm.at[pl.ds((c - ch0) * t_nodes, t_nodes)],
                        osems[b]).wait()

                for j in range(t_nodes):
                    def rbody(r, acc):
                        return tuple(
                            acc[k] + bufs[b][j * ns + r,
                                             pl.ds(k * _LANES, _LANES)]
                            for k in range(ncol)
                        )
                    acc = lax.fori_loop(
                        0, ns, rbody,
                        tuple(jnp.zeros((_LANES,), jnp.float32)
                              for _ in range(ncol)),
                        unroll=8,
                    )
                    for k in range(ncol):
                        obufs[b][j, pl.ds(k * _LANES, _LANES)] = acc[k]
                pltpu.async_copy(
                    obufs[b],
                    nsum_hbm.at[pl.ds((c - ch0) * t_nodes, t_nodes)],
                    osems[b])

        return carry

    lax.fori_loop(0, niter // _NBUF, outer, 0)

    for b in range(_NBUF):
        pltpu.make_async_copy(
            obufs[b], nsum_hbm.at[pl.ds(0, t_nodes)], osems[b]).wait()


def _sc_segsum(neighbor_feats, n, ns, d, ch0, t_nodes):
    body = functools.partial(_sc_segsum_body, t_nodes=t_nodes, ns=ns, d=d,
                             n=n, ch0=ch0)
    mesh = plsc.VectorSubcoreMesh(core_axis_name="c", subcore_axis_name="s")
    rows = t_nodes * ns
    return pl.kernel(
        body,
        out_type=jax.ShapeDtypeStruct((n - ch0 * t_nodes, d), jnp.float32),
        mesh=mesh,
        scratch_types=(
            [pltpu.VMEM((rows, d), jnp.float32) for _ in range(_NBUF)]
            + [pltpu.VMEM((t_nodes, d), jnp.float32) for _ in range(_NBUF)]
            + [pltpu.SemaphoreType.DMA for _ in range(2 * _NBUF)]
        ),
    )(neighbor_feats)


def _fused_body(nf_ref, nb_ref, w_ref, out_ref, *, inv_ns):
    nsum = jnp.sum(nb_ref[...], axis=1)
    d = nf_ref.shape[1]
    acc = jnp.dot(nf_ref[...], w_ref[:d, :], preferred_element_type=jnp.float32)
    acc += jnp.dot(nsum * inv_ns, w_ref[d:, :],
                   preferred_element_type=jnp.float32)
    out_ref[...] = jnp.maximum(acc, 0.0)


def _dense_body(prev_ref, nf_ref, nsum_ref, w_ref, out_ref, *, inv_ns):
    del prev_ref
    d = nf_ref.shape[1]
    acc = jnp.dot(nf_ref[...], w_ref[:d, :], preferred_element_type=jnp.float32)
    acc += jnp.dot(nsum_ref[...] * inv_ns, w_ref[d:, :],
                   preferred_element_type=jnp.float32)
    out_ref[...] = jnp.maximum(acc, 0.0)


def kernel(node_feats, neighbor_feats, weight, node_count):
    n, d = node_feats.shape
    ns = neighbor_feats.shape[0] // n
    e = weight.shape[1]
    s = _SPLIT
    blk = _BLK
    inv_ns = 1.0 / ns

    nb3 = neighbor_feats.reshape(n, ns, d)

    def fused_call(o, k, prev):
        if prev is None:
            args = (node_feats, nb3, weight)
            specs = []
            aliases = {}
            body = _fused_body
        else:
            args = (prev, node_feats, nb3, weight)
            specs = [pl.BlockSpec(memory_space=pl.ANY)]
            aliases = {0: 0}
            body = lambda p, *r: _fused_body(*r, inv_ns=inv_ns)
        return pl.pallas_call(
            (functools.partial(_fused_body, inv_ns=inv_ns)
             if prev is None else body),
            grid=(k,),
            in_specs=specs + [
                pl.BlockSpec((blk, d), lambda i: (i + o, 0)),
                pl.BlockSpec((blk, ns, d), lambda i: (i + o, 0, 0)),
                pl.BlockSpec((2 * d, e), lambda i: (0, 0)),
            ],
            out_specs=pl.BlockSpec((blk, e), lambda i: (i + o, 0)),
            out_shape=jax.ShapeDtypeStruct((n, e), jnp.float32),
            input_output_aliases=aliases,
            compiler_params=pltpu.CompilerParams(
                dimension_semantics=("arbitrary",),
            ),
        )(*args)

    fused_a = fused_call(0, 1, None)

    nb_dep, _ = lax.optimization_barrier((neighbor_feats, fused_a))
    nsum = _sc_segsum(nb_dep, n, ns, d, ch0=s // _TNODES,
                      t_nodes=_TNODES)

    fused = fused_call(1, s // blk - 1, fused_a)

    off = s // blk
    out = pl.pallas_call(
        functools.partial(_dense_body, inv_ns=inv_ns),
        grid=((n - s) // blk,),
        in_specs=[
            pl.BlockSpec(memory_space=pl.ANY),
            pl.BlockSpec((blk, d), lambda i: (i + off, 0)),
            pl.BlockSpec((blk, d), lambda i: (i, 0)),
            pl.BlockSpec((2 * d, e), lambda i: (0, 0)),
        ],
        out_specs=pl.BlockSpec((blk, e), lambda i: (i + off, 0)),
        out_shape=jax.ShapeDtypeStruct((n, e), jnp.float32),
        input_output_aliases={0: 0},
        compiler_params=pltpu.CompilerParams(
            dimension_semantics=("arbitrary",),
        ),
    )(fused, node_feats, nsum, weight)
    return out


# speedup vs baseline: 1.2251x; 1.2251x over previous
import functools

import jax
import jax.numpy as jnp
from jax import lax
from jax.experimental import pallas as pl
from jax.experimental.pallas import tpu as pltpu
from jax.experimental.pallas import tpu_sc as plsc

_NC = 2
_NSC = 16
_NW = _NC * _NSC
_LANES = 16

_BLK = 400
_SPLIT = 9200


def _sc_segsum_body(nb_hbm, nsum_hbm, buf0, buf1, obuf, sem0, sem1,
                    *, t_nodes, ns, d, n, ch0):
    nch = n // t_nodes
    rows = t_nodes * ns
    ncol = d // _LANES
    count = nch - ch0
    niter = (count + _NW - 1) // _NW
    niter += niter % 2
    bufs = (buf0, buf1)
    sems = (sem0, sem1)
    wid = lax.axis_index("c") * _NSC + lax.axis_index("s")

    c0 = ch0 + wid

    @pl.when(c0 < nch)
    def _():
        pltpu.async_copy(nb_hbm.at[pl.ds(c0 * rows, rows)], bufs[0], sems[0])

    def outer(i, carry):
        for b in range(2):
            c = ch0 + wid + _NW * (2 * i + b)

            @pl.when(c < nch)
            def _(c=c, b=b):
                pltpu.make_async_copy(
                    nb_hbm.at[pl.ds(c * rows, rows)], bufs[b], sems[b]).wait()
                cn = c + _NW

                @pl.when(cn < nch)
                def _():
                    pltpu.async_copy(
                        nb_hbm.at[pl.ds(cn * rows, rows)], bufs[1 - b],
                        sems[1 - b])

                for j in range(t_nodes):
                    def rbody(r, acc):
                        return tuple(
                            acc[k] + bufs[b][j * ns + r,
                                             pl.ds(k * _LANES, _LANES)]
                            for k in range(ncol)
                        )
                    acc = lax.fori_loop(
                        0, ns, rbody,
                        tuple(jnp.zeros((_LANES,), jnp.float32)
                              for _ in range(ncol)),
                        unroll=4,
                    )
                    for k in range(ncol):
                        obuf[j, pl.ds(k * _LANES, _LANES)] = acc[k]
                pltpu.sync_copy(
                    obuf, nsum_hbm.at[pl.ds((c - ch0) * t_nodes, t_nodes)])

        return carry

    lax.fori_loop(0, niter // 2, outer, 0)


def _sc_segsum(neighbor_feats, n, ns, d, ch0, t_nodes):
    body = functools.partial(_sc_segsum_body, t_nodes=t_nodes, ns=ns, d=d,
                             n=n, ch0=ch0)
    mesh = plsc.VectorSubcoreMesh(core_axis_name="c", subcore_axis_name="s")
    rows = t_nodes * ns
    return pl.kernel(
        body,
        out_type=jax.ShapeDtypeStruct((n - ch0 * t_nodes, d), jnp.float32),
        mesh=mesh,
        scratch_types=[
            pltpu.VMEM((rows, d), jnp.float32),
            pltpu.VMEM((rows, d), jnp.float32),
            pltpu.VMEM((t_nodes, d), jnp.float32),
            pltpu.SemaphoreType.DMA,
            pltpu.SemaphoreType.DMA,
        ],
    )(neighbor_feats)


def _fused_body(nf_ref, nb_ref, w_ref, out_ref, *, inv_ns):
    nsum = jnp.sum(nb_ref[...], axis=1)
    d = nf_ref.shape[1]
    acc = jnp.dot(nf_ref[...], w_ref[:d, :], preferred_element_type=jnp.float32)
    acc += jnp.dot(nsum * inv_ns, w_ref[d:, :],
                   preferred_element_type=jnp.float32)
    out_ref[...] = jnp.maximum(acc, 0.0)


def _dense_body(prev_ref, nf_ref, nsum_ref, w_ref, out_ref, *, inv_ns):
    del prev_ref
    d = nf_ref.shape[1]
    acc = jnp.dot(nf_ref[...], w_ref[:d, :], preferred_element_type=jnp.float32)
    acc += jnp.dot(nsum_ref[...] * inv_ns, w_ref[d:, :],
                   preferred_element_type=jnp.float32)
    out_ref[...] = jnp.maximum(acc, 0.0)


def kernel(node_feats, neighbor_feats, weight, node_count):
    n, d = node_feats.shape
    ns = neighbor_feats.shape[0] // n
    e = weight.shape[1]
    s = _SPLIT
    blk = _BLK
    inv_ns = 1.0 / ns

    nb3 = neighbor_feats.reshape(n, ns, d)

    if s > 0:
        fused = pl.pallas_call(
            functools.partial(_fused_body, inv_ns=inv_ns),
            grid=(s // blk,),
            in_specs=[
                pl.BlockSpec((blk, d), lambda i: (i, 0)),
                pl.BlockSpec((blk, ns, d), lambda i: (i, 0, 0)),
                pl.BlockSpec((2 * d, e), lambda i: (0, 0)),
            ],
            out_specs=pl.BlockSpec((blk, e), lambda i: (i, 0)),
            out_shape=jax.ShapeDtypeStruct((n, e), jnp.float32),
            compiler_params=pltpu.CompilerParams(
                dimension_semantics=("arbitrary",),
            ),
        )(node_feats, nb3, weight)
    else:
        fused = jnp.zeros((n, e), jnp.float32)

    nsum = _sc_segsum(neighbor_feats, n, ns, d, ch0=s // 8, t_nodes=8)

    off = s // blk
    out = pl.pallas_call(
        functools.partial(_dense_body, inv_ns=inv_ns),
        grid=((n - s) // blk,),
        in_specs=[
            pl.BlockSpec(memory_space=pl.ANY),
            pl.BlockSpec((blk, d), lambda i: (i + off, 0)),
            pl.BlockSpec((blk, d), lambda i: (i, 0)),
            pl.BlockSpec((2 * d, e), lambda i: (0, 0)),
        ],
        out_specs=pl.BlockSpec((blk, e), lambda i: (i + off, 0)),
        out_shape=jax.ShapeDtypeStruct((n, e), jnp.float32),
        input_output_aliases={0: 0},
        compiler_params=pltpu.CompilerParams(
            dimension_semantics=("arbitrary",),
        ),
    )(fused, node_feats, nsum, weight)
    return out


# speedup vs baseline: 1.2283x; 1.0026x over previous
import functools

import jax
import jax.numpy as jnp
from jax import lax
from jax.experimental import pallas as pl
from jax.experimental.pallas import tpu as pltpu
from jax.experimental.pallas import tpu_sc as plsc

_NC = 2
_NSC = 16
_NW = _NC * _NSC
_LANES = 16

_BLK = 400
_SPLIT = 9600
_BLKF = 800


def _sc_segsum_body(nb_hbm, nsum_hbm, buf0, buf1, obuf, sem0, sem1,
                    *, t_nodes, ns, d, n, ch0):
    nch = n // t_nodes
    rows = t_nodes * ns
    ncol = d // _LANES
    count = nch - ch0
    niter = (count + _NW - 1) // _NW
    niter += niter % 2
    bufs = (buf0, buf1)
    sems = (sem0, sem1)
    wid = lax.axis_index("c") * _NSC + lax.axis_index("s")

    c0 = ch0 + wid

    @pl.when(c0 < nch)
    def _():
        pltpu.async_copy(nb_hbm.at[pl.ds(c0 * rows, rows)], bufs[0], sems[0])

    def outer(i, carry):
        for b in range(2):
            c = ch0 + wid + _NW * (2 * i + b)

            @pl.when(c < nch)
            def _(c=c, b=b):
                pltpu.make_async_copy(
                    nb_hbm.at[pl.ds(c * rows, rows)], bufs[b], sems[b]).wait()
                cn = c + _NW

                @pl.when(cn < nch)
                def _():
                    pltpu.async_copy(
                        nb_hbm.at[pl.ds(cn * rows, rows)], bufs[1 - b],
                        sems[1 - b])

                for j in range(t_nodes):
                    def rbody(r, acc):
                        return tuple(
                            acc[k] + bufs[b][j * ns + r,
                                             pl.ds(k * _LANES, _LANES)]
                            for k in range(ncol)
                        )
                    acc = lax.fori_loop(
                        0, ns, rbody,
                        tuple(jnp.zeros((_LANES,), jnp.float32)
                              for _ in range(ncol)),
                        unroll=4,
                    )
                    for k in range(ncol):
                        obuf[j, pl.ds(k * _LANES, _LANES)] = acc[k]
                pltpu.sync_copy(
                    obuf, nsum_hbm.at[pl.ds((c - ch0) * t_nodes, t_nodes)])

        return carry

    lax.fori_loop(0, niter // 2, outer, 0)


def _sc_segsum(neighbor_feats, n, ns, d, ch0, t_nodes):
    body = functools.partial(_sc_segsum_body, t_nodes=t_nodes, ns=ns, d=d,
                             n=n, ch0=ch0)
    mesh = plsc.VectorSubcoreMesh(core_axis_name="c", subcore_axis_name="s")
    rows = t_nodes * ns
    return pl.kernel(
        body,
        out_type=jax.ShapeDtypeStruct((n - ch0 * t_nodes, d), jnp.float32),
        mesh=mesh,
        scratch_types=[
            pltpu.VMEM((rows, d), jnp.float32),
            pltpu.VMEM((rows, d), jnp.float32),
            pltpu.VMEM((t_nodes, d), jnp.float32),
            pltpu.SemaphoreType.DMA,
            pltpu.SemaphoreType.DMA,
        ],
    )(neighbor_feats)


def _fused_body(nf_ref, nb_ref, w_ref, out_ref, *, inv_ns):
    nsum = jnp.sum(nb_ref[...], axis=1)
    d = nf_ref.shape[1]
    acc = jnp.dot(nf_ref[...], w_ref[:d, :], preferred_element_type=jnp.float32)
    acc += jnp.dot(nsum * inv_ns, w_ref[d:, :],
                   preferred_element_type=jnp.float32)
    out_ref[...] = jnp.maximum(acc, 0.0)


def _dense_body(prev_ref, nf_ref, nsum_ref, w_ref, out_ref, *, inv_ns):
    del prev_ref
    d = nf_ref.shape[1]
    acc = jnp.dot(nf_ref[...], w_ref[:d, :], preferred_element_type=jnp.float32)
    acc += jnp.dot(nsum_ref[...] * inv_ns, w_ref[d:, :],
                   preferred_element_type=jnp.float32)
    out_ref[...] = jnp.maximum(acc, 0.0)


def kernel(node_feats, neighbor_feats, weight, node_count):
    n, d = node_feats.shape
    ns = neighbor_feats.shape[0] // n
    e = weight.shape[1]
    s = _SPLIT
    blk = _BLK
    inv_ns = 1.0 / ns

    nb3 = neighbor_feats.reshape(n, ns, d)

    if s > 0:
        bf = _BLKF
        fused = pl.pallas_call(
            functools.partial(_fused_body, inv_ns=inv_ns),
            grid=(s // bf,),
            in_specs=[
                pl.BlockSpec((bf, d), lambda i: (i, 0)),
                pl.BlockSpec((bf, ns, d), lambda i: (i, 0, 0)),
                pl.BlockSpec((2 * d, e), lambda i: (0, 0)),
            ],
            out_specs=pl.BlockSpec((bf, e), lambda i: (i, 0)),
            out_shape=jax.ShapeDtypeStruct((n, e), jnp.float32),
            compiler_params=pltpu.CompilerParams(
                dimension_semantics=("arbitrary",),
            ),
        )(node_feats, nb3, weight)
    else:
        fused = jnp.zeros((n, e), jnp.float32)

    nsum = _sc_segsum(neighbor_feats, n, ns, d, ch0=s // 8, t_nodes=8)

    off = s // blk
    out = pl.pallas_call(
        functools.partial(_dense_body, inv_ns=inv_ns),
        grid=((n - s) // blk,),
        in_specs=[
            pl.BlockSpec(memory_space=pl.ANY),
            pl.BlockSpec((blk, d), lambda i: (i + off, 0)),
            pl.BlockSpec((blk, d), lambda i: (i, 0)),
            pl.BlockSpec((2 * d, e), lambda i: (0, 0)),
        ],
        out_specs=pl.BlockSpec((blk, e), lambda i: (i + off, 0)),
        out_shape=jax.ShapeDtypeStruct((n, e), jnp.float32),
        input_output_aliases={0: 0},
        compiler_params=pltpu.CompilerParams(
            dimension_semantics=("arbitrary",),
        ),
    )(fused, node_feats, nsum, weight)
    return out
